# Initial kernel scaffold; baseline (speedup 1.0000x reference)
import jax
import jax.numpy as jnp
from jax.experimental import pallas as pl


def kernel(token_ids, weights, table):
    raise NotImplementedError("write your pallas kernel here")


# speedup vs baseline: 2.4501x; 2.4501x over previous
import functools

import jax
import jax.numpy as jnp
from jax import lax
from jax.experimental import pallas as pl
from jax.experimental.pallas import tpu as pltpu
from jax.experimental.pallas import tpu_sc as plsc

B = 16384
K = 50
D = 64
NC = 2
NS = 16
NW = NC * NS
SENT_PER_W = B // NW
S = 16
CHUNKS = SENT_PER_W // S
ROWS = S * K
LANES = 16
DV = D // LANES
KP = 64
KG = (K + LANES - 1) // LANES


def _sc_body(ids_hbm, w_hbm, table_hbm, out_hbm, idx_v, w_v, rows_v, out_v, sem):
    wid = lax.axis_index("s") * NC + lax.axis_index("c")
    base_s = wid * SENT_PER_W

    def chunk_body(c, carry):
        s0 = base_s + c * S
        f0 = s0 * K
        pltpu.sync_copy(ids_hbm.at[pl.ds(f0, ROWS)], idx_v)
        pltpu.sync_copy(w_hbm.at[pl.ds(s0 * KP, S * KP)], w_v)
        pltpu.async_copy(table_hbm.at[idx_v], rows_v, sem).wait()

        def sent_body(s, carry2):
            r0 = s * K
            wb = s * KP
            zero = jnp.zeros((LANES,), jnp.float32)
            accs = [zero] * DV
            for g in range(KG):
                cnt = min(LANES, K - g * LANES)
                w16 = w_v[pl.ds(wb + g * LANES, LANES)]
                for j in range(cnt):
                    wv = lax.broadcast(w16[j], (LANES,))
                    fi = r0 + g * LANES + j
                    for d in range(DV):
                        accs[d] = accs[d] + wv * rows_v[fi, pl.ds(d * LANES, LANES)]
            inv_k = jnp.float32(1.0 / K)
            for d in range(DV):
                out_v[s, pl.ds(d * LANES, LANES)] = accs[d] * inv_k
            return carry2

        lax.fori_loop(0, S, sent_body, 0)
        pltpu.sync_copy(out_v, out_hbm.at[pl.ds(s0, S)])
        return carry

    lax.fori_loop(0, CHUNKS, chunk_body, 0)


@jax.jit
def kernel(token_ids, weights, table):
    ids_flat = token_ids.reshape(-1).astype(jnp.int32)
    w_flat = jnp.pad(weights, ((0, 0), (0, KP - K))).reshape(-1)
    mesh = plsc.VectorSubcoreMesh(core_axis_name="c", subcore_axis_name="s")
    out = pl.kernel(
        _sc_body,
        out_type=jax.ShapeDtypeStruct((B, D), jnp.float32),
        mesh=mesh,
        scratch_types=[
            pltpu.VMEM((ROWS,), jnp.int32),
            pltpu.VMEM((S * KP,), jnp.float32),
            pltpu.VMEM((ROWS, D), jnp.float32),
            pltpu.VMEM((S, D), jnp.float32),
            pltpu.SemaphoreType.DMA,
        ],
        compiler_params=pltpu.CompilerParams(use_tc_tiling_on_sc=False),
    )(ids_flat, w_flat, table)
    return out
